# baseline (speedup 1.0000x reference)
import functools

import jax
import jax.numpy as jnp
from jax.experimental import pallas as pl

E = 8
TOPK = 2


def _moe_block(xb_ref, wrt_ref, wg2t_ref, c01_ref, wgate_ref, wup_ref,
               wdown_ref, out_ref, logits_ref):
    x = xb_ref[...]
    tb = x.shape[0]

    t = jax.lax.dot_general(x, wrt_ref[...], (((1,), (0,)), ((), ())),
                            preferred_element_type=jnp.float32)
    r = jnp.tanh(t[:, 0:64])
    a0 = t[:, 64:65]
    a1 = t[:, 65:66]
    logits = jax.lax.dot_general(r, wg2t_ref[...], (((1,), (0,)), ((), ())),
                                 preferred_element_type=jnp.float32)
    logits_ref[...] = logits

    m = jnp.max(logits, axis=1, keepdims=True)
    ex = jnp.exp(logits - m)
    p = ex / jnp.sum(ex, axis=1, keepdims=True)
    idx = jax.lax.broadcasted_iota(jnp.int32, (tb, E), 1)
    m1 = jnp.max(p, axis=1, keepdims=True)
    i1 = jnp.min(jnp.where(p == m1, idx, E), axis=1, keepdims=True)
    oh1 = idx == i1
    p2 = jnp.where(oh1, -1.0, p)
    m2 = jnp.max(p2, axis=1, keepdims=True)
    i2 = jnp.min(jnp.where(p2 == m2, idx, E), axis=1, keepdims=True)
    sel = oh1 | (idx == i2)
    wz = jnp.where(sel & (idx != E - 1), p, 0.0)
    we = wz / jnp.sum(wz, axis=1, keepdims=True)

    def ew(k):
        return jnp.sum(jnp.where(idx == k, we, 0.0), axis=1, keepdims=True)

    w_copy = ew(0)
    w_c0 = ew(2)
    w_c1 = ew(3)
    w_shared = jnp.sum(jnp.where(idx >= 4, we, 0.0), axis=1, keepdims=True)

    c0 = c01_ref[0:1, :]
    c1 = c01_ref[1:2, :]
    s0 = jax.nn.sigmoid(a0)
    s1 = jax.nn.sigmoid(a1)
    ce = (w_c0 * s0 + w_c1 * s1) * x \
        + (w_c0 * (1.0 - s0)) * c0 + (w_c1 * (1.0 - s1)) * c1

    x16 = x.astype(jnp.bfloat16)
    g = jax.lax.dot_general(x16, wgate_ref[...], (((1,), (0,)), ((), ())),
                            preferred_element_type=jnp.float32)
    u = jax.lax.dot_general(x16, wup_ref[...], (((1,), (0,)), ((), ())),
                            preferred_element_type=jnp.float32)
    h = (g * jax.nn.sigmoid(g) * u).astype(jnp.bfloat16)
    y = jax.lax.dot_general(h, wdown_ref[...], (((1,), (0,)), ((), ())),
                            preferred_element_type=jnp.float32)

    out_ref[...] = w_copy * x + ce + w_shared * y


@functools.partial(jax.jit, static_argnames=())
def _run(x, wrt, wg2t, c01, wgate_t, wup_t, wdown_t):
    T, D = x.shape
    FF = wgate_t.shape[1]
    TB = 256
    grid = (T // TB,)
    out, logits = pl.pallas_call(
        _moe_block,
        grid=grid,
        in_specs=[
            pl.BlockSpec((TB, D), lambda i: (i, 0)),
            pl.BlockSpec((D, 128), lambda i: (0, 0)),
            pl.BlockSpec((64, E), lambda i: (0, 0)),
            pl.BlockSpec((2, D), lambda i: (0, 0)),
            pl.BlockSpec((D, FF), lambda i: (0, 0)),
            pl.BlockSpec((D, FF), lambda i: (0, 0)),
            pl.BlockSpec((FF, D), lambda i: (0, 0)),
        ],
        out_specs=[
            pl.BlockSpec((TB, D), lambda i: (i, 0)),
            pl.BlockSpec((TB, E), lambda i: (i, 0)),
        ],
        out_shape=[
            jax.ShapeDtypeStruct((T, D), jnp.float32),
            jax.ShapeDtypeStruct((T, E), jnp.float32),
        ],
    )(x, wrt, wg2t, c01, wgate_t, wup_t, wdown_t)
    return out, logits


def kernel(hidden_sates, W_g1, W_g2, const0, wg0, const1, wg1, W_gate,
           W_up, W_down):
    b, s, d = hidden_sates.shape
    x = hidden_sates.reshape(-1, d).astype(jnp.float32)
    wrt = jnp.zeros((d, 128), jnp.float32)
    wrt = wrt.at[:, 0:64].set(W_g1.T)
    wrt = wrt.at[:, 64].set(wg0[0] - wg0[1])
    wrt = wrt.at[:, 65].set(wg1[0] - wg1[1])
    wg2t = W_g2.T
    c01 = jnp.stack([const0, const1], axis=0)
    wgate_t = W_gate.T.astype(jnp.bfloat16)
    wup_t = W_up.T.astype(jnp.bfloat16)
    wdown_t = W_down.T.astype(jnp.bfloat16)
    out, logits = _run(x, wrt, wg2t, c01, wgate_t, wup_t, wdown_t)
    return out.reshape(b, s, d), logits


# speedup vs baseline: 1.2247x; 1.2247x over previous
import functools

import jax
import jax.numpy as jnp
from jax.experimental import pallas as pl

E = 8
TOPK = 2


def _moe_block(xb_ref, wrt_ref, wg2t_ref, c01_ref, wgate_ref, wup_ref,
               wdown_ref, out_ref, logits_ref):
    x = xb_ref[...]
    tb = x.shape[0]

    t = jax.lax.dot_general(x, wrt_ref[...], (((1,), (0,)), ((), ())),
                            preferred_element_type=jnp.float32)
    r = jnp.tanh(t[:, 0:64])
    a0 = t[:, 64:65]
    a1 = t[:, 65:66]
    logits = jax.lax.dot_general(r, wg2t_ref[...], (((1,), (0,)), ((), ())),
                                 preferred_element_type=jnp.float32)
    logits_ref[...] = logits

    m = jnp.max(logits, axis=1, keepdims=True)
    ex = jnp.exp(logits - m)
    p = ex / jnp.sum(ex, axis=1, keepdims=True)
    idx = jax.lax.broadcasted_iota(jnp.int32, (tb, E), 1)
    m1 = jnp.max(p, axis=1, keepdims=True)
    i1 = jnp.min(jnp.where(p == m1, idx, E), axis=1, keepdims=True)
    oh1 = idx == i1
    p2 = jnp.where(oh1, -1.0, p)
    m2 = jnp.max(p2, axis=1, keepdims=True)
    i2 = jnp.min(jnp.where(p2 == m2, idx, E), axis=1, keepdims=True)
    sel = oh1 | (idx == i2)
    wz = jnp.where(sel & (idx != E - 1), p, 0.0)
    we = wz / jnp.sum(wz, axis=1, keepdims=True)

    def ew(k):
        return jnp.sum(jnp.where(idx == k, we, 0.0), axis=1, keepdims=True)

    w_copy = ew(0)
    w_c0 = ew(2)
    w_c1 = ew(3)
    w_shared = jnp.sum(jnp.where(idx >= 4, we, 0.0), axis=1, keepdims=True)

    c0 = c01_ref[0:1, :]
    c1 = c01_ref[1:2, :]
    s0 = jax.nn.sigmoid(a0)
    s1 = jax.nn.sigmoid(a1)
    ce = (w_c0 * s0 + w_c1 * s1) * x \
        + (w_c0 * (1.0 - s0)) * c0 + (w_c1 * (1.0 - s1)) * c1

    g = jax.lax.dot_general(x, wgate_ref[...], (((1,), (1,)), ((), ())),
                            preferred_element_type=jnp.float32)
    u = jax.lax.dot_general(x, wup_ref[...], (((1,), (1,)), ((), ())),
                            preferred_element_type=jnp.float32)
    h = g * jax.nn.sigmoid(g) * u
    y = jax.lax.dot_general(h, wdown_ref[...], (((1,), (1,)), ((), ())),
                            preferred_element_type=jnp.float32)

    out_ref[...] = w_copy * x + ce + w_shared * y


@functools.partial(jax.jit, static_argnames=())
def _run(x, wrt, wg2t, c01, wgate, wup, wdown):
    T, D = x.shape
    FF = wgate.shape[0]
    TB = 256
    grid = (T // TB,)
    out, logits = pl.pallas_call(
        _moe_block,
        grid=grid,
        in_specs=[
            pl.BlockSpec((TB, D), lambda i: (i, 0)),
            pl.BlockSpec((D, 128), lambda i: (0, 0)),
            pl.BlockSpec((64, E), lambda i: (0, 0)),
            pl.BlockSpec((2, D), lambda i: (0, 0)),
            pl.BlockSpec((FF, D), lambda i: (0, 0)),
            pl.BlockSpec((FF, D), lambda i: (0, 0)),
            pl.BlockSpec((D, FF), lambda i: (0, 0)),
        ],
        out_specs=[
            pl.BlockSpec((TB, D), lambda i: (i, 0)),
            pl.BlockSpec((TB, E), lambda i: (i, 0)),
        ],
        out_shape=[
            jax.ShapeDtypeStruct((T, D), jnp.float32),
            jax.ShapeDtypeStruct((T, E), jnp.float32),
        ],
    )(x, wrt, wg2t, c01, wgate, wup, wdown)
    return out, logits


def kernel(hidden_sates, W_g1, W_g2, const0, wg0, const1, wg1, W_gate,
           W_up, W_down):
    b, s, d = hidden_sates.shape
    x = hidden_sates.reshape(-1, d).astype(jnp.float32)
    wrt = jnp.zeros((d, 128), jnp.float32)
    wrt = wrt.at[:, 0:64].set(W_g1.T)
    wrt = wrt.at[:, 64].set(wg0[0] - wg0[1])
    wrt = wrt.at[:, 65].set(wg1[0] - wg1[1])
    wg2t = W_g2.T
    c01 = jnp.stack([const0, const1], axis=0)
    out, logits = _run(x, wrt, wg2t, c01, W_gate, W_up, W_down)
    return out.reshape(b, s, d), logits


# speedup vs baseline: 1.5109x; 1.2337x over previous
import functools

import jax
import jax.numpy as jnp
from jax.experimental import pallas as pl
from jax.experimental.pallas import tpu as pltpu

E = 8
TOPK = 2
TB = 512
ACT_DTYPE = jnp.bfloat16
FFC = 512


def _moe_block(xb_ref, wg1_ref, wg2_ref, c0_ref, c1_ref, wg0_ref, wg1c_ref,
               wgate_ref, wup_ref, wdown_ref, out_ref, logits_ref):
    x = xb_ref[...]
    tb = x.shape[0]

    t = jax.lax.dot_general(x, wg1_ref[...], (((1,), (1,)), ((), ())),
                            preferred_element_type=jnp.float32)
    r = jnp.tanh(t)
    logits = jax.lax.dot_general(r, wg2_ref[...], (((1,), (1,)), ((), ())),
                                 preferred_element_type=jnp.float32)
    logits_ref[...] = logits
    ta0 = jax.lax.dot_general(x, wg0_ref[...], (((1,), (1,)), ((), ())),
                              preferred_element_type=jnp.float32)
    ta1 = jax.lax.dot_general(x, wg1c_ref[...], (((1,), (1,)), ((), ())),
                              preferred_element_type=jnp.float32)
    a0 = ta0[:, 0:1] - ta0[:, 1:2]
    a1 = ta1[:, 0:1] - ta1[:, 1:2]

    idx = jax.lax.broadcasted_iota(jnp.int32, (tb, E), 1)
    m1 = jnp.max(logits, axis=1, keepdims=True)
    i1 = jnp.min(jnp.where(logits == m1, idx, E), axis=1, keepdims=True)
    oh1 = idx == i1
    l2 = jnp.where(oh1, -jnp.inf, logits)
    m2 = jnp.max(l2, axis=1, keepdims=True)
    i2 = jnp.min(jnp.where(l2 == m2, idx, E), axis=1, keepdims=True)
    sel = oh1 | (idx == i2)
    wz = jnp.where(sel & (idx != E - 1), jnp.exp(logits - m1), 0.0)
    we = wz / jnp.sum(wz, axis=1, keepdims=True)

    def ew(k):
        return jnp.sum(jnp.where(idx == k, we, 0.0), axis=1, keepdims=True)

    w_copy = ew(0)
    w_c0 = ew(2)
    w_c1 = ew(3)
    w_shared = jnp.sum(jnp.where(idx >= 4, we, 0.0), axis=1, keepdims=True)

    c0 = c0_ref[...]
    c1 = c1_ref[...]
    s0 = jax.nn.sigmoid(a0)
    s1 = jax.nn.sigmoid(a1)
    x_coef = w_copy + w_c0 * s0 + w_c1 * s1
    b0 = w_c0 * (1.0 - s0)
    b1 = w_c1 * (1.0 - s1)

    g = jax.lax.dot_general(x, wgate_ref[...], (((1,), (1,)), ((), ())),
                            preferred_element_type=jnp.float32)
    u = jax.lax.dot_general(x, wup_ref[...], (((1,), (1,)), ((), ())),
                            preferred_element_type=jnp.float32)
    h = (g * jax.nn.sigmoid(g) * u).astype(ACT_DTYPE)
    y = jax.lax.dot_general(h, wdown_ref[...], (((1,), (1,)), ((), ())),
                            preferred_element_type=jnp.float32)

    out_ref[...] = x_coef * x + b0 * c0 + b1 * c1 + w_shared * y


@functools.partial(jax.jit, static_argnames=())
def _run(x, wg1, wg2, c0, c1, wg0, wg1c, wgate, wup, wdown):
    T, D = x.shape
    FF = wgate.shape[0]
    grid = (T // TB,)
    out, logits = pl.pallas_call(
        _moe_block,
        grid=grid,
        in_specs=[
            pl.BlockSpec((TB, D), lambda i: (i, 0)),
            pl.BlockSpec((E * 8, D), lambda i: (0, 0)),
            pl.BlockSpec((E, E * 8), lambda i: (0, 0)),
            pl.BlockSpec((1, D), lambda i: (0, 0)),
            pl.BlockSpec((1, D), lambda i: (0, 0)),
            pl.BlockSpec((2, D), lambda i: (0, 0)),
            pl.BlockSpec((2, D), lambda i: (0, 0)),
            pl.BlockSpec((FF, D), lambda i: (0, 0)),
            pl.BlockSpec((FF, D), lambda i: (0, 0)),
            pl.BlockSpec((D, FF), lambda i: (0, 0)),
        ],
        out_specs=[
            pl.BlockSpec((TB, D), lambda i: (i, 0)),
            pl.BlockSpec((TB, E), lambda i: (i, 0)),
        ],
        out_shape=[
            jax.ShapeDtypeStruct((T, D), jnp.float32),
            jax.ShapeDtypeStruct((T, E), jnp.float32),
        ],
    )(x, wg1, wg2, c0, c1, wg0, wg1c, wgate, wup, wdown)
    return out, logits


def kernel(hidden_sates, W_g1, W_g2, const0, wg0, const1, wg1, W_gate,
           W_up, W_down):
    b, s, d = hidden_sates.shape
    x = hidden_sates.reshape(-1, d).astype(jnp.float32)
    out, logits = _run(x, W_g1, W_g2, const0.reshape(1, d),
                       const1.reshape(1, d), wg0, wg1, W_gate, W_up, W_down)
    return out.reshape(b, s, d), logits
